# baseline (speedup 1.0000x reference)
import functools

import jax
import jax.numpy as jnp
from jax import lax
from jax.experimental import pallas as pl
from jax.experimental.pallas import tpu as pltpu
from jax.experimental.pallas import tpu_sc as plsc

N = 10000
E = 320000
D = 128
C = 32
FC = 512
NOUT = 10

NC = 2
NS = 16
NW = NC * NS
CH = 128
EPW = 10240
NCH = EPW // CH
E_PAD = EPW * NW
PAD_ROW = N
N_ACC = 10112
RPS = N_ACC // NS

def _deg_body(e0_hbm, e1_hbm, zeros_hbm, ones_hbm, out_hbm,
              dst_v, ones_v, acc_sh):
    c = lax.axis_index("c")
    s = lax.axis_index("s")
    pltpu.sync_copy(zeros_hbm.at[pl.ds(s * RPS, RPS)],
                    acc_sh.at[pl.ds(s * RPS, RPS)])

    @pl.when(c == 0)
    def _copy0():
        pltpu.sync_copy(e0_hbm.at[1, s], dst_v.at[pl.ds(0, NCH0)])

    @pl.when(c == 1)
    def _copy1():
        pltpu.sync_copy(e1_hbm.at[1, s], dst_v.at[pl.ds(0, NCH1)])

    pltpu.sync_copy(ones_hbm, ones_v)
    plsc.subcore_barrier()
    nch = jnp.where(c == 0, NCH0, NCH1)

    @pl.loop(0, nch)
    def _chunk(j):
        pltpu.sync_copy(ones_v, acc_sh.at[dst_v.at[j]], add=True)

    plsc.subcore_barrier()
    pltpu.sync_copy(acc_sh.at[pl.ds(s * RPS, RPS)],
                    out_hbm.at[c, pl.ds(s * RPS, RPS)])


NBUF = 8
NCH0 = 40
NCH1 = 120
EC0 = NS * NCH0 * CH
NCHMX = max(NCH0, NCH1)


def _prop_body(g_hbm, e0_hbm, e1_hbm, zeros_hbm, out_hbm,
               src_v, dst_v, bufs, acc_sh, gsems, ssems):
    c = lax.axis_index("c")
    s = lax.axis_index("s")
    pltpu.sync_copy(zeros_hbm.at[pl.ds(s * RPS, RPS)],
                    acc_sh.at[pl.ds(s * RPS, RPS)])

    @pl.when(c == 0)
    def _copy0():
        pltpu.sync_copy(e0_hbm.at[0, s], src_v.at[pl.ds(0, NCH0)])
        pltpu.sync_copy(e0_hbm.at[1, s], dst_v.at[pl.ds(0, NCH0)])

    @pl.when(c == 1)
    def _copy1():
        pltpu.sync_copy(e1_hbm.at[0, s], src_v.at[pl.ds(0, NCH1)])
        pltpu.sync_copy(e1_hbm.at[1, s], dst_v.at[pl.ds(0, NCH1)])

    plsc.subcore_barrier()
    nch = jnp.where(c == 0, NCH0, NCH1)

    for b in range(NBUF):
        pltpu.async_copy(g_hbm.at[src_v.at[b]], bufs.at[b], gsems.at[b])

    @pl.loop(0, nch, step=NBUF)
    def _group(j):
        for b in range(NBUF):
            pltpu.make_async_copy(
                g_hbm.at[src_v.at[j + b]], bufs.at[b], gsems.at[b]).wait()
            pltpu.async_copy(
                bufs.at[b], acc_sh.at[dst_v.at[j + b]], ssems.at[b], add=True)
        for b in range(NBUF):
            pltpu.make_async_copy(
                bufs.at[b], acc_sh.at[dst_v.at[j + b]], ssems.at[b]).wait()
            pltpu.async_copy(
                g_hbm.at[src_v.at[lax.rem(j + NBUF + b, nch)]],
                bufs.at[b], gsems.at[b])

    for b in range(NBUF):
        pltpu.make_async_copy(
            g_hbm.at[src_v.at[b]], bufs.at[b], gsems.at[b]).wait()

    plsc.subcore_barrier()
    pltpu.sync_copy(acc_sh.at[pl.ds(s * RPS, RPS)],
                    out_hbm.at[c, pl.ds(s * RPS, RPS)])


def _tc1_body(degp_ref, x_ref, w1_ref, g1_ref, dinv_ref):
    deg = degp_ref[0, :N, 0:1] + degp_ref[1, :N, 0:1] + 1.0
    dinv = lax.rsqrt(jnp.maximum(deg, 1.0))
    xw = jnp.dot(x_ref[...], w1_ref[...], preferred_element_type=jnp.float32)
    g1_ref[:N, :] = dinv * xw
    g1_ref[N:, :] = jnp.zeros((N_ACC - N, C), jnp.float32)
    dinv_ref[:N, :] = dinv
    dinv_ref[N:, :] = jnp.zeros((N_ACC - N, 1), jnp.float32)


def _elu(x):
    return jnp.where(x > 0, x, jnp.exp(jnp.minimum(x, 0.0)) - 1.0)


def _tc2_body(p_ref, g1_ref, dinv_ref, b1_ref, w2_ref, g2_ref):
    dinv = dinv_ref[:N, :]
    pre = (dinv * (p_ref[0, :N, :] + p_ref[1, :N, :] + g1_ref[:N, :])
           + b1_ref[...][None, :])
    h1 = _elu(pre)
    hw = jnp.dot(h1, w2_ref[...], preferred_element_type=jnp.float32)
    g2_ref[:N, :] = dinv * hw
    g2_ref[N:, :] = jnp.zeros((N_ACC - N, C), jnp.float32)


def _tc3_body(p_ref, g2_ref, dinv_ref, b2_ref, h2_ref):
    dinv = dinv_ref[:N, :]
    pre = (dinv * (p_ref[0, :N, :] + p_ref[1, :N, :] + g2_ref[:N, :])
           + b2_ref[...][None, :])
    h2_ref[...] = _elu(pre)


@functools.lru_cache(maxsize=None)
def _build_sc_kernels():
    mesh = plsc.VectorSubcoreMesh(
        core_axis_name="c", subcore_axis_name="s",
        num_cores=NC, num_subcores=NS)
    params = pltpu.CompilerParams(use_tc_tiling_on_sc=False)
    deg_k = pl.kernel(
        _deg_body,
        out_type=jax.ShapeDtypeStruct((NC, N_ACC, 16), jnp.float32),
        mesh=mesh,
        compiler_params=params,
        scratch_types=[
            pltpu.VMEM((NCHMX, CH), jnp.int32),
            pltpu.VMEM((CH, 16), jnp.float32),
            pltpu.VMEM_SHARED((N_ACC, 16), jnp.float32),
        ],
    )
    prop_k = pl.kernel(
        _prop_body,
        out_type=jax.ShapeDtypeStruct((NC, N_ACC, C), jnp.float32),
        mesh=mesh,
        compiler_params=params,
        scratch_types=[
            pltpu.VMEM((NCHMX, CH), jnp.int32),
            pltpu.VMEM((NCHMX, CH), jnp.int32),
            pltpu.VMEM((NBUF, CH, C), jnp.float32),
            pltpu.VMEM_SHARED((N_ACC, C), jnp.float32),
            pltpu.SemaphoreType.DMA((NBUF,)),
            pltpu.SemaphoreType.DMA((NBUF,)),
        ],
    )
    return deg_k, prop_k


def _deg_call(e0, e1, zeros16, ones16):
    return _build_sc_kernels()[0](e0, e1, zeros16, ones16)


def _prop_call(g, e0, e1, zeros32):
    return _build_sc_kernels()[1](g, e0, e1, zeros32)


KB = 6400
KSTEPS = (N * C) // KB


def _matvec_body(flat_ref, wfc1_ref, bfc1_ref, wfc2_ref, bfc2_ref,
                 out_ref, acc_ref):
    k = pl.program_id(0)

    @pl.when(k == 0)
    def _init():
        acc_ref[...] = bfc1_ref[...]

    acc_ref[...] += jnp.dot(flat_ref[...], wfc1_ref[...],
                            preferred_element_type=jnp.float32)

    @pl.when(k == KSTEPS - 1)
    def _head():
        fc = jnp.maximum(acc_ref[...], 0.0)
        logits = jnp.dot(fc, wfc2_ref[...],
                         preferred_element_type=jnp.float32) + bfc2_ref[...]
        m = jnp.max(logits, axis=-1, keepdims=True)
        e = jnp.exp(logits - m)
        out_ref[...] = e / jnp.sum(e, axis=-1, keepdims=True)


def kernel(x, edge_index, W1, b1, W2, b2, Wfc1, bfc1, Wfc2, bfc2):
    pad = jnp.full((2, E_PAD - E), PAD_ROW, dtype=jnp.int32)
    flat = jnp.concatenate([edge_index, pad], axis=1)
    e0 = flat[:, :EC0].reshape(2, NS, NCH0, CH)
    e1 = flat[:, EC0:].reshape(2, NS, NCH1, CH)
    zeros16 = jnp.zeros((N_ACC, 16), jnp.float32)
    zeros32 = jnp.zeros((N_ACC, C), jnp.float32)
    ones16 = jnp.ones((CH, 16), jnp.float32)

    degp = _deg_call(e0, e1, zeros16, ones16)

    g1, dinv = pl.pallas_call(
        _tc1_body,
        out_shape=[
            jax.ShapeDtypeStruct((N_ACC, C), jnp.float32),
            jax.ShapeDtypeStruct((N_ACC, 1), jnp.float32),
        ],
    )(degp, x, W1)

    p1 = _prop_call(g1, e0, e1, zeros32)

    g2 = pl.pallas_call(
        _tc2_body,
        out_shape=jax.ShapeDtypeStruct((N_ACC, C), jnp.float32),
    )(p1, g1, dinv, b1, W2)

    p2 = _prop_call(g2, e0, e1, zeros32)

    h2 = pl.pallas_call(
        _tc3_body,
        out_shape=jax.ShapeDtypeStruct((N, C), jnp.float32),
    )(p2, g2, dinv, b2)

    flat = h2.reshape(1, N * C)

    out = pl.pallas_call(
        _matvec_body,
        grid=(KSTEPS,),
        in_specs=[
            pl.BlockSpec((1, KB), lambda k: (0, k)),
            pl.BlockSpec((KB, FC), lambda k: (k, 0)),
            pl.BlockSpec((1, FC), lambda k: (0, 0)),
            pl.BlockSpec((FC, NOUT), lambda k: (0, 0)),
            pl.BlockSpec((1, NOUT), lambda k: (0, 0)),
        ],
        out_specs=pl.BlockSpec((1, NOUT), lambda k: (0, 0)),
        out_shape=jax.ShapeDtypeStruct((1, NOUT), jnp.float32),
        scratch_shapes=[pltpu.VMEM((1, FC), jnp.float32)],
    )(flat, Wfc1, bfc1.reshape(1, FC), Wfc2, bfc2.reshape(1, NOUT))

    return out


# speedup vs baseline: 1.0384x; 1.0384x over previous
import functools

import jax
import jax.numpy as jnp
from jax import lax
from jax.experimental import pallas as pl
from jax.experimental.pallas import tpu as pltpu
from jax.experimental.pallas import tpu_sc as plsc

N = 10000
E = 320000
D = 128
C = 32
FC = 512
NOUT = 10

NC = 2
NS = 16
NW = NC * NS
CH = 128
EPW = 10240
NCH = EPW // CH
E_PAD = EPW * NW
PAD_ROW = N
N_ACC = 10112
RPS = N_ACC // NS

def _deg_body(e0_hbm, e1_hbm, zeros_hbm, ones_hbm, out_hbm,
              dst_v, ones_v, acc_sh):
    c = lax.axis_index("c")
    s = lax.axis_index("s")
    pltpu.sync_copy(zeros_hbm.at[pl.ds(s * RPS, RPS)],
                    acc_sh.at[pl.ds(s * RPS, RPS)])

    @pl.when(c == 0)
    def _copy0():
        pltpu.sync_copy(e0_hbm.at[1, s], dst_v.at[pl.ds(0, NCH0)])

    @pl.when(c == 1)
    def _copy1():
        pltpu.sync_copy(e1_hbm.at[1, s], dst_v.at[pl.ds(0, NCH1)])

    pltpu.sync_copy(ones_hbm, ones_v)
    plsc.subcore_barrier()
    nch = jnp.where(c == 0, NCH0, NCH1)

    @pl.loop(0, nch)
    def _chunk(j):
        pltpu.sync_copy(ones_v, acc_sh.at[dst_v.at[j]], add=True)

    plsc.subcore_barrier()
    pltpu.sync_copy(acc_sh.at[pl.ds(s * RPS, RPS)],
                    out_hbm.at[c, pl.ds(s * RPS, RPS)])


NBUF = 8
NCH0 = 120
NCH1 = 40
EC0 = NS * NCH0 * CH
NCHMX = max(NCH0, NCH1)


def _prop_body(g_hbm, e0_hbm, e1_hbm, zeros_hbm, out_hbm,
               src_v, dst_v, bufs, acc_sh, gsems, ssems):
    c = lax.axis_index("c")
    s = lax.axis_index("s")
    pltpu.sync_copy(zeros_hbm.at[pl.ds(s * RPS, RPS)],
                    acc_sh.at[pl.ds(s * RPS, RPS)])

    @pl.when(c == 0)
    def _copy0():
        pltpu.sync_copy(e0_hbm.at[0, s], src_v.at[pl.ds(0, NCH0)])
        pltpu.sync_copy(e0_hbm.at[1, s], dst_v.at[pl.ds(0, NCH0)])

    @pl.when(c == 1)
    def _copy1():
        pltpu.sync_copy(e1_hbm.at[0, s], src_v.at[pl.ds(0, NCH1)])
        pltpu.sync_copy(e1_hbm.at[1, s], dst_v.at[pl.ds(0, NCH1)])

    plsc.subcore_barrier()
    nch = jnp.where(c == 0, NCH0, NCH1)

    for b in range(NBUF):
        pltpu.async_copy(g_hbm.at[src_v.at[b]], bufs.at[b], gsems.at[b])

    @pl.loop(0, nch, step=NBUF)
    def _group(j):
        for b in range(NBUF):
            pltpu.make_async_copy(
                g_hbm.at[src_v.at[j + b]], bufs.at[b], gsems.at[b]).wait()
            pltpu.async_copy(
                bufs.at[b], acc_sh.at[dst_v.at[j + b]], ssems.at[b], add=True)
        for b in range(NBUF):
            pltpu.make_async_copy(
                bufs.at[b], acc_sh.at[dst_v.at[j + b]], ssems.at[b]).wait()
            pltpu.async_copy(
                g_hbm.at[src_v.at[lax.rem(j + NBUF + b, nch)]],
                bufs.at[b], gsems.at[b])

    for b in range(NBUF):
        pltpu.make_async_copy(
            g_hbm.at[src_v.at[b]], bufs.at[b], gsems.at[b]).wait()

    plsc.subcore_barrier()
    pltpu.sync_copy(acc_sh.at[pl.ds(s * RPS, RPS)],
                    out_hbm.at[c, pl.ds(s * RPS, RPS)])


def _tc1_body(degp_ref, x_ref, w1_ref, g1_ref, dinv_ref):
    deg = degp_ref[0, :N, 0:1] + degp_ref[1, :N, 0:1] + 1.0
    dinv = lax.rsqrt(jnp.maximum(deg, 1.0))
    xw = jnp.dot(x_ref[...], w1_ref[...], preferred_element_type=jnp.float32)
    g1_ref[:N, :] = dinv * xw
    g1_ref[N:, :] = jnp.zeros((N_ACC - N, C), jnp.float32)
    dinv_ref[:N, :] = dinv
    dinv_ref[N:, :] = jnp.zeros((N_ACC - N, 1), jnp.float32)


def _elu(x):
    return jnp.where(x > 0, x, jnp.exp(jnp.minimum(x, 0.0)) - 1.0)


def _tc2_body(p_ref, g1_ref, dinv_ref, b1_ref, w2_ref, g2_ref):
    dinv = dinv_ref[:N, :]
    pre = (dinv * (p_ref[0, :N, :] + p_ref[1, :N, :] + g1_ref[:N, :])
           + b1_ref[...][None, :])
    h1 = _elu(pre)
    hw = jnp.dot(h1, w2_ref[...], preferred_element_type=jnp.float32)
    g2_ref[:N, :] = dinv * hw
    g2_ref[N:, :] = jnp.zeros((N_ACC - N, C), jnp.float32)


def _tc3_body(p_ref, g2_ref, dinv_ref, b2_ref, h2_ref):
    dinv = dinv_ref[:N, :]
    pre = (dinv * (p_ref[0, :N, :] + p_ref[1, :N, :] + g2_ref[:N, :])
           + b2_ref[...][None, :])
    h2_ref[...] = _elu(pre)


@functools.lru_cache(maxsize=None)
def _build_sc_kernels():
    mesh = plsc.VectorSubcoreMesh(
        core_axis_name="c", subcore_axis_name="s",
        num_cores=NC, num_subcores=NS)
    params = pltpu.CompilerParams(use_tc_tiling_on_sc=False)
    deg_k = pl.kernel(
        _deg_body,
        out_type=jax.ShapeDtypeStruct((NC, N_ACC, 16), jnp.float32),
        mesh=mesh,
        compiler_params=params,
        scratch_types=[
            pltpu.VMEM((NCHMX, CH), jnp.int32),
            pltpu.VMEM((CH, 16), jnp.float32),
            pltpu.VMEM_SHARED((N_ACC, 16), jnp.float32),
        ],
    )
    prop_k = pl.kernel(
        _prop_body,
        out_type=jax.ShapeDtypeStruct((NC, N_ACC, C), jnp.float32),
        mesh=mesh,
        compiler_params=params,
        scratch_types=[
            pltpu.VMEM((NCHMX, CH), jnp.int32),
            pltpu.VMEM((NCHMX, CH), jnp.int32),
            pltpu.VMEM((NBUF, CH, C), jnp.float32),
            pltpu.VMEM_SHARED((N_ACC, C), jnp.float32),
            pltpu.SemaphoreType.DMA((NBUF,)),
            pltpu.SemaphoreType.DMA((NBUF,)),
        ],
    )
    return deg_k, prop_k


def _deg_call(e0, e1, zeros16, ones16):
    return _build_sc_kernels()[0](e0, e1, zeros16, ones16)


def _prop_call(g, e0, e1, zeros32):
    return _build_sc_kernels()[1](g, e0, e1, zeros32)


KB = 6400
KSTEPS = (N * C) // KB


def _matvec_body(flat_ref, wfc1_ref, bfc1_ref, wfc2_ref, bfc2_ref,
                 out_ref, acc_ref):
    k = pl.program_id(0)

    @pl.when(k == 0)
    def _init():
        acc_ref[...] = bfc1_ref[...]

    acc_ref[...] += jnp.dot(flat_ref[...], wfc1_ref[...],
                            preferred_element_type=jnp.float32)

    @pl.when(k == KSTEPS - 1)
    def _head():
        fc = jnp.maximum(acc_ref[...], 0.0)
        logits = jnp.dot(fc, wfc2_ref[...],
                         preferred_element_type=jnp.float32) + bfc2_ref[...]
        m = jnp.max(logits, axis=-1, keepdims=True)
        e = jnp.exp(logits - m)
        out_ref[...] = e / jnp.sum(e, axis=-1, keepdims=True)


def kernel(x, edge_index, W1, b1, W2, b2, Wfc1, bfc1, Wfc2, bfc2):
    pad = jnp.full((2, E_PAD - E), PAD_ROW, dtype=jnp.int32)
    flat = jnp.concatenate([edge_index, pad], axis=1)
    e0 = flat[:, :EC0].reshape(2, NS, NCH0, CH)
    e1 = flat[:, EC0:].reshape(2, NS, NCH1, CH)
    zeros16 = jnp.zeros((N_ACC, 16), jnp.float32)
    zeros32 = jnp.zeros((N_ACC, C), jnp.float32)
    ones16 = jnp.ones((CH, 16), jnp.float32)

    degp = _deg_call(e0, e1, zeros16, ones16)

    g1, dinv = pl.pallas_call(
        _tc1_body,
        out_shape=[
            jax.ShapeDtypeStruct((N_ACC, C), jnp.float32),
            jax.ShapeDtypeStruct((N_ACC, 1), jnp.float32),
        ],
    )(degp, x, W1)

    p1 = _prop_call(g1, e0, e1, zeros32)

    g2 = pl.pallas_call(
        _tc2_body,
        out_shape=jax.ShapeDtypeStruct((N_ACC, C), jnp.float32),
    )(p1, g1, dinv, b1, W2)

    p2 = _prop_call(g2, e0, e1, zeros32)

    h2 = pl.pallas_call(
        _tc3_body,
        out_shape=jax.ShapeDtypeStruct((N, C), jnp.float32),
    )(p2, g2, dinv, b2)

    flat = h2.reshape(1, N * C)

    out = pl.pallas_call(
        _matvec_body,
        grid=(KSTEPS,),
        in_specs=[
            pl.BlockSpec((1, KB), lambda k: (0, k)),
            pl.BlockSpec((KB, FC), lambda k: (k, 0)),
            pl.BlockSpec((1, FC), lambda k: (0, 0)),
            pl.BlockSpec((FC, NOUT), lambda k: (0, 0)),
            pl.BlockSpec((1, NOUT), lambda k: (0, 0)),
        ],
        out_specs=pl.BlockSpec((1, NOUT), lambda k: (0, 0)),
        out_shape=jax.ShapeDtypeStruct((1, NOUT), jnp.float32),
        scratch_shapes=[pltpu.VMEM((1, FC), jnp.float32)],
    )(flat, Wfc1, bfc1.reshape(1, FC), Wfc2, bfc2.reshape(1, NOUT))

    return out


# speedup vs baseline: 1.4350x; 1.3819x over previous
import functools

import jax
import jax.numpy as jnp
from jax import lax
from jax.experimental import pallas as pl
from jax.experimental.pallas import tpu as pltpu
from jax.experimental.pallas import tpu_sc as plsc

N = 10000
E = 320000
D = 128
C = 32
FC = 512
NOUT = 10

NC = 2
NS = 16
NW = NC * NS
CH = 128
EPW = 10240
NCH = EPW // CH
E_PAD = EPW * NW
PAD_ROW = N
N_ACC = 10112
RPS = N_ACC // NS

def _deg_body(e0_hbm, e1_hbm, zeros_hbm, ones_hbm, out_hbm,
              dst_v, ones_v, acc_sh):
    c = lax.axis_index("c")
    s = lax.axis_index("s")
    pltpu.sync_copy(zeros_hbm.at[pl.ds(s * RPS, RPS)],
                    acc_sh.at[pl.ds(s * RPS, RPS)])

    @pl.when(c == 0)
    def _copy0():
        pltpu.sync_copy(e0_hbm.at[1, s], dst_v.at[pl.ds(0, NCH0)])

    @pl.when(c == 1)
    def _copy1():
        pltpu.sync_copy(e1_hbm.at[1, s], dst_v.at[pl.ds(0, NCH1)])

    pltpu.sync_copy(ones_hbm, ones_v)
    plsc.subcore_barrier()
    nch = jnp.where(c == 0, NCH0, NCH1)

    @pl.loop(0, nch)
    def _chunk(j):
        pltpu.sync_copy(ones_v, acc_sh.at[dst_v.at[j]], add=True)

    plsc.subcore_barrier()
    pltpu.sync_copy(acc_sh.at[pl.ds(s * RPS, RPS)],
                    out_hbm.at[c, pl.ds(s * RPS, RPS)])


NBUF = 8
NCH0 = 80
NCH1 = 80
EC0 = NS * NCH0 * CH
NCHMX = max(NCH0, NCH1)


def _prop_body(g_hbm, e0_hbm, e1_hbm, zeros_hbm, out_hbm,
               src_v, dst_v, bufs, acc_sh, g_sh, gsems, ssems):
    c = lax.axis_index("c")
    s = lax.axis_index("s")
    pltpu.sync_copy(zeros_hbm.at[pl.ds(s * RPS, RPS)],
                    acc_sh.at[pl.ds(s * RPS, RPS)])
    pltpu.sync_copy(g_hbm.at[pl.ds(s * RPS, RPS)],
                    g_sh.at[pl.ds(s * RPS, RPS)])

    @pl.when(c == 0)
    def _copy0():
        pltpu.sync_copy(e0_hbm.at[0, s], src_v.at[pl.ds(0, NCH0)])
        pltpu.sync_copy(e0_hbm.at[1, s], dst_v.at[pl.ds(0, NCH0)])

    @pl.when(c == 1)
    def _copy1():
        pltpu.sync_copy(e1_hbm.at[0, s], src_v.at[pl.ds(0, NCH1)])
        pltpu.sync_copy(e1_hbm.at[1, s], dst_v.at[pl.ds(0, NCH1)])

    plsc.subcore_barrier()
    nch = jnp.where(c == 0, NCH0, NCH1)

    for b in range(NBUF):
        pltpu.async_copy(g_sh.at[src_v.at[b]], bufs.at[b], gsems.at[b])

    @pl.loop(0, nch, step=NBUF)
    def _group(j):
        for b in range(NBUF):
            pltpu.make_async_copy(
                g_sh.at[src_v.at[j + b]], bufs.at[b], gsems.at[b]).wait()
            pltpu.async_copy(
                bufs.at[b], acc_sh.at[dst_v.at[j + b]], ssems.at[b], add=True)
        for b in range(NBUF):
            pltpu.make_async_copy(
                bufs.at[b], acc_sh.at[dst_v.at[j + b]], ssems.at[b]).wait()
            pltpu.async_copy(
                g_sh.at[src_v.at[lax.rem(j + NBUF + b, nch)]],
                bufs.at[b], gsems.at[b])

    for b in range(NBUF):
        pltpu.make_async_copy(
            g_sh.at[src_v.at[b]], bufs.at[b], gsems.at[b]).wait()

    plsc.subcore_barrier()
    pltpu.sync_copy(acc_sh.at[pl.ds(s * RPS, RPS)],
                    out_hbm.at[c, pl.ds(s * RPS, RPS)])


def _tc1_body(degp_ref, x_ref, w1_ref, g1_ref, dinv_ref):
    deg = degp_ref[0, :N, 0:1] + degp_ref[1, :N, 0:1] + 1.0
    dinv = lax.rsqrt(jnp.maximum(deg, 1.0))
    xw = jnp.dot(x_ref[...], w1_ref[...], preferred_element_type=jnp.float32)
    g1_ref[:N, :] = dinv * xw
    g1_ref[N:, :] = jnp.zeros((N_ACC - N, C), jnp.float32)
    dinv_ref[:N, :] = dinv
    dinv_ref[N:, :] = jnp.zeros((N_ACC - N, 1), jnp.float32)


def _elu(x):
    return jnp.where(x > 0, x, jnp.exp(jnp.minimum(x, 0.0)) - 1.0)


def _tc2_body(p_ref, g1_ref, dinv_ref, b1_ref, w2_ref, g2_ref):
    dinv = dinv_ref[:N, :]
    pre = (dinv * (p_ref[0, :N, :] + p_ref[1, :N, :] + g1_ref[:N, :])
           + b1_ref[...][None, :])
    h1 = _elu(pre)
    hw = jnp.dot(h1, w2_ref[...], preferred_element_type=jnp.float32)
    g2_ref[:N, :] = dinv * hw
    g2_ref[N:, :] = jnp.zeros((N_ACC - N, C), jnp.float32)


def _tc3_body(p_ref, g2_ref, dinv_ref, b2_ref, h2_ref):
    dinv = dinv_ref[:N, :]
    pre = (dinv * (p_ref[0, :N, :] + p_ref[1, :N, :] + g2_ref[:N, :])
           + b2_ref[...][None, :])
    h2_ref[...] = _elu(pre)


@functools.lru_cache(maxsize=None)
def _build_sc_kernels():
    mesh = plsc.VectorSubcoreMesh(
        core_axis_name="c", subcore_axis_name="s",
        num_cores=NC, num_subcores=NS)
    params = pltpu.CompilerParams(use_tc_tiling_on_sc=False)
    deg_k = pl.kernel(
        _deg_body,
        out_type=jax.ShapeDtypeStruct((NC, N_ACC, 16), jnp.float32),
        mesh=mesh,
        compiler_params=params,
        scratch_types=[
            pltpu.VMEM((NCHMX, CH), jnp.int32),
            pltpu.VMEM((CH, 16), jnp.float32),
            pltpu.VMEM_SHARED((N_ACC, 16), jnp.float32),
        ],
    )
    prop_k = pl.kernel(
        _prop_body,
        out_type=jax.ShapeDtypeStruct((NC, N_ACC, C), jnp.float32),
        mesh=mesh,
        compiler_params=params,
        scratch_types=[
            pltpu.VMEM((NCHMX, CH), jnp.int32),
            pltpu.VMEM((NCHMX, CH), jnp.int32),
            pltpu.VMEM((NBUF, CH, C), jnp.float32),
            pltpu.VMEM_SHARED((N_ACC, C), jnp.float32),
            pltpu.VMEM_SHARED((N_ACC, C), jnp.float32),
            pltpu.SemaphoreType.DMA((NBUF,)),
            pltpu.SemaphoreType.DMA((NBUF,)),
        ],
    )
    return deg_k, prop_k


def _deg_call(e0, e1, zeros16, ones16):
    return _build_sc_kernels()[0](e0, e1, zeros16, ones16)


def _prop_call(g, e0, e1, zeros32):
    return _build_sc_kernels()[1](g, e0, e1, zeros32)


KB = 6400
KSTEPS = (N * C) // KB


def _matvec_body(flat_ref, wfc1_ref, bfc1_ref, wfc2_ref, bfc2_ref,
                 out_ref, acc_ref):
    k = pl.program_id(0)

    @pl.when(k == 0)
    def _init():
        acc_ref[...] = bfc1_ref[...]

    acc_ref[...] += jnp.dot(flat_ref[...], wfc1_ref[...],
                            preferred_element_type=jnp.float32)

    @pl.when(k == KSTEPS - 1)
    def _head():
        fc = jnp.maximum(acc_ref[...], 0.0)
        logits = jnp.dot(fc, wfc2_ref[...],
                         preferred_element_type=jnp.float32) + bfc2_ref[...]
        m = jnp.max(logits, axis=-1, keepdims=True)
        e = jnp.exp(logits - m)
        out_ref[...] = e / jnp.sum(e, axis=-1, keepdims=True)


def kernel(x, edge_index, W1, b1, W2, b2, Wfc1, bfc1, Wfc2, bfc2):
    pad = jnp.full((2, E_PAD - E), PAD_ROW, dtype=jnp.int32)
    flat = jnp.concatenate([edge_index, pad], axis=1)
    e0 = flat[:, :EC0].reshape(2, NS, NCH0, CH)
    e1 = flat[:, EC0:].reshape(2, NS, NCH1, CH)
    zeros16 = jnp.zeros((N_ACC, 16), jnp.float32)
    zeros32 = jnp.zeros((N_ACC, C), jnp.float32)
    ones16 = jnp.ones((CH, 16), jnp.float32)

    degp = _deg_call(e0, e1, zeros16, ones16)

    g1, dinv = pl.pallas_call(
        _tc1_body,
        out_shape=[
            jax.ShapeDtypeStruct((N_ACC, C), jnp.float32),
            jax.ShapeDtypeStruct((N_ACC, 1), jnp.float32),
        ],
    )(degp, x, W1)

    p1 = _prop_call(g1, e0, e1, zeros32)

    g2 = pl.pallas_call(
        _tc2_body,
        out_shape=jax.ShapeDtypeStruct((N_ACC, C), jnp.float32),
    )(p1, g1, dinv, b1, W2)

    p2 = _prop_call(g2, e0, e1, zeros32)

    h2 = pl.pallas_call(
        _tc3_body,
        out_shape=jax.ShapeDtypeStruct((N, C), jnp.float32),
    )(p2, g2, dinv, b2)

    flat = h2.reshape(1, N * C)

    out = pl.pallas_call(
        _matvec_body,
        grid=(KSTEPS,),
        in_specs=[
            pl.BlockSpec((1, KB), lambda k: (0, k)),
            pl.BlockSpec((KB, FC), lambda k: (k, 0)),
            pl.BlockSpec((1, FC), lambda k: (0, 0)),
            pl.BlockSpec((FC, NOUT), lambda k: (0, 0)),
            pl.BlockSpec((1, NOUT), lambda k: (0, 0)),
        ],
        out_specs=pl.BlockSpec((1, NOUT), lambda k: (0, 0)),
        out_shape=jax.ShapeDtypeStruct((1, NOUT), jnp.float32),
        scratch_shapes=[pltpu.VMEM((1, FC), jnp.float32)],
    )(flat, Wfc1, bfc1.reshape(1, FC), Wfc2, bfc2.reshape(1, NOUT))

    return out
